# baseline (speedup 1.0000x reference)
import functools

import jax
import jax.numpy as jnp
from jax import lax
from jax.experimental import pallas as pl
from jax.experimental.pallas import tpu as pltpu
from jax.experimental.pallas import tpu_sc as plsc

_NC = 2
_NS = 16
_NW = _NC * _NS


def _dequant_body(w_ref, s_ref, o_ref):
    o_ref[...] = w_ref[...].astype(jnp.bfloat16) * s_ref[...]


@functools.partial(jax.jit, static_argnames=("blk",))
def _dequant_table(weight, scaler, blk=4000):
    n, d = weight.shape
    return pl.pallas_call(
        _dequant_body,
        grid=(n // blk,),
        in_specs=[
            pl.BlockSpec((blk, d), lambda i: (i, 0)),
            pl.BlockSpec((1, d), lambda i: (0, 0)),
        ],
        out_specs=pl.BlockSpec((blk, d), lambda i: (i, 0)),
        out_shape=jax.ShapeDtypeStruct((n, d), jnp.bfloat16),
    )(weight, scaler.reshape(1, d))


def _make_gather(num_idx, table_shape, chunk):
    n, d = table_shape
    per_w = num_idx // _NW
    n_chunks = per_w // chunk
    assert per_w % chunk == 0 and num_idx % _NW == 0

    @functools.partial(
        pl.kernel,
        out_type=jax.ShapeDtypeStruct((num_idx, d), jnp.int32),
        mesh=plsc.VectorSubcoreMesh(
            core_axis_name="c", subcore_axis_name="s",
            num_cores=_NC, num_subcores=_NS),
        scratch_types=[
            pltpu.VMEM((per_w,), jnp.int32),
            pltpu.VMEM((chunk, d), jnp.int32),
            pltpu.VMEM((chunk, d), jnp.int32),
            pltpu.SemaphoreType.DMA,
            pltpu.SemaphoreType.DMA,
        ],
        compiler_params=pltpu.CompilerParams(use_tc_tiling_on_sc=False),
    )
    def gather_kernel(table_hbm, idx_hbm, out_hbm, idx_v, buf0, buf1,
                      sem0, sem1):
        wid = lax.axis_index("s") * _NC + lax.axis_index("c")
        base = wid * per_w
        pltpu.sync_copy(idx_hbm.at[pl.ds(base, per_w)], idx_v)

        bufs = (buf0, buf1)
        sems = (sem0, sem1)

        def start(i):
            return pltpu.async_copy(
                table_hbm.at[idx_v.at[pl.ds(i * chunk, chunk)]],
                bufs[i % 2], sems[i % 2])

        pend = [None] * n_chunks
        pend[0] = start(0)
        for i in range(n_chunks):
            if i + 1 < n_chunks:
                pend[i + 1] = start(i + 1)
            pend[i].wait()
            pltpu.sync_copy(bufs[i % 2],
                            out_hbm.at[pl.ds(base + i * chunk, chunk)])

    return gather_kernel


def kernel(input, weight, weight_scaler):
    b, h = input.shape
    n, d = weight.shape
    table = _dequant_table(weight, weight_scaler)
    table_i32 = lax.bitcast_convert_type(
        table.reshape(n, d // 2, 2), jnp.int32)
    flat_idx = input.reshape(-1)
    gather = _make_gather(b * h, (n, d // 2), chunk=800)
    out_i32 = gather(table_i32, flat_idx)
    out = lax.bitcast_convert_type(out_i32, jnp.bfloat16)
    return out.reshape(b, h, d)


# speedup vs baseline: 1.9688x; 1.9688x over previous
import functools

import jax
import jax.numpy as jnp
from jax import lax
from jax.experimental import pallas as pl
from jax.experimental.pallas import tpu as pltpu
from jax.experimental.pallas import tpu_sc as plsc

_NC = 2
_NS = 16
_NW = _NC * _NS
_LANES = 16


def _dequant_body(w_ref, s_ref, o_ref):
    o_ref[...] = w_ref[...].astype(jnp.bfloat16) * s_ref[...]


def _dequant_table(weight, scaler, blk=4000):
    n, d = weight.shape
    return pl.pallas_call(
        _dequant_body,
        grid=(n // blk,),
        in_specs=[
            pl.BlockSpec((blk, d), lambda i: (i, 0)),
            pl.BlockSpec((1, d), lambda i: (0, 0)),
        ],
        out_specs=pl.BlockSpec((blk, d), lambda i: (i, 0)),
        out_shape=jax.ShapeDtypeStruct((n, d), jnp.bfloat16),
    )(weight, scaler.reshape(1, d))


def _make_gather(b, h, d, n, nb):
    dw = d // 2
    num_idx = b * h
    per_w = num_idx // _NW
    batches_per_w = b // _NW
    chunk = nb * h
    n_chunks = batches_per_w // nb
    assert b % _NW == 0 and batches_per_w % nb == 0

    @functools.partial(
        pl.kernel,
        out_type=jax.ShapeDtypeStruct((b, h, d), jnp.bfloat16),
        mesh=plsc.VectorSubcoreMesh(
            core_axis_name="c", subcore_axis_name="s",
            num_cores=_NC, num_subcores=_NS),
        scratch_types=[
            pltpu.VMEM((per_w,), jnp.int32),
            pltpu.VMEM((chunk, dw), jnp.int32),
            pltpu.VMEM((chunk, dw), jnp.int32),
            pltpu.VMEM((nb, h, d), jnp.bfloat16),
            pltpu.VMEM((nb, h, d), jnp.bfloat16),
            pltpu.SemaphoreType.DMA,
            pltpu.SemaphoreType.DMA,
            pltpu.SemaphoreType.DMA,
            pltpu.SemaphoreType.DMA,
        ],
        compiler_params=pltpu.CompilerParams(
            use_tc_tiling_on_sc=False, needs_layout_passes=False),
    )
    def gather_kernel(table_hbm, idx_hbm, out_hbm, idx_v, g0, g1, s0, s1,
                      gsem0, gsem1, ssem0, ssem1):
        wid = lax.axis_index("s") * _NC + lax.axis_index("c")
        base = wid * per_w
        batch0 = wid * batches_per_w
        pltpu.sync_copy(idx_hbm.at[pl.ds(base, per_w)], idx_v)

        gbufs = (g0, g1)
        sbufs = (s0, s1)
        gsems = (gsem0, gsem1)
        ssems = (ssem0, ssem1)

        def start_gather(i):
            return pltpu.async_copy(
                table_hbm.at[idx_v.at[pl.ds(i * chunk, chunk)]],
                gbufs[i % 2], gsems[i % 2])

        def stage_and_store(i):
            g = gbufs[i % 2]
            s = sbufs[i % 2]
            def copy_row(r, _):
                q = r // h
                row = r - q * h
                for k in range(dw // _LANES):
                    w32 = g[r, pl.ds(k * _LANES, _LANES)]
                    s[q, row, pl.ds(k * 2 * _LANES, 2 * _LANES)] = (
                        plsc.bitcast(w32, jnp.bfloat16))
                return ()
            lax.fori_loop(0, chunk, copy_row, (), unroll=4)
            return pltpu.async_copy(
                s, out_hbm.at[pl.ds(batch0 + i * nb, nb)], ssems[i % 2])

        gather_pend = [None] * n_chunks
        store_pend = [None] * n_chunks
        gather_pend[0] = start_gather(0)
        for i in range(n_chunks):
            if i + 1 < n_chunks:
                gather_pend[i + 1] = start_gather(i + 1)
            gather_pend[i].wait()
            if i >= 2:
                store_pend[i - 2].wait()
            store_pend[i] = stage_and_store(i)
        for i in range(max(n_chunks - 2, 0), n_chunks):
            store_pend[i].wait()

    return gather_kernel


def kernel(input, weight, weight_scaler):
    b, h = input.shape
    n, d = weight.shape
    table = _dequant_table(weight, weight_scaler)
    table32 = lax.bitcast_convert_type(
        table.reshape(n, d // 2, 2), jnp.int32)
    flat_idx = input.reshape(-1)
    gather = _make_gather(b, h, d, n, nb=2)
    return gather(table32, flat_idx)


# speedup vs baseline: 3.8797x; 1.9706x over previous
import functools

import jax
import jax.numpy as jnp
from jax import lax
from jax.experimental import pallas as pl
from jax.experimental.pallas import tpu as pltpu
from jax.experimental.pallas import tpu_sc as plsc

_NC = 2
_NS = 16
_NW = _NC * _NS
_LANES = 16


def _dequant_body(w_ref, s_ref, o_ref):
    vals = w_ref[...].astype(jnp.bfloat16) * s_ref[...]
    u = pltpu.bitcast(vals, jnp.uint16).astype(jnp.uint32)
    o_ref[...] = (u | (u << 16)).astype(jnp.int32)


def _dequant_table(weight, scaler, blk=4000):
    n, d = weight.shape
    return pl.pallas_call(
        _dequant_body,
        grid=(n // blk,),
        in_specs=[
            pl.BlockSpec((blk, d), lambda i: (i, 0)),
            pl.BlockSpec((1, d), lambda i: (0, 0)),
        ],
        out_specs=pl.BlockSpec((blk, d), lambda i: (i, 0)),
        out_shape=jax.ShapeDtypeStruct((n, d), jnp.int32),
    )(weight, scaler.reshape(1, d))


def _make_gather(b, h, d, n, nb):
    num_idx = b * h
    per_w = num_idx // _NW
    batches_per_w = b // _NW
    chunk = nb * h
    n_chunks = batches_per_w // nb
    assert b % _NW == 0 and batches_per_w % nb == 0

    @functools.partial(
        pl.kernel,
        out_type=jax.ShapeDtypeStruct((b, h, d), jnp.bfloat16),
        mesh=plsc.VectorSubcoreMesh(
            core_axis_name="c", subcore_axis_name="s",
            num_cores=_NC, num_subcores=_NS),
        scratch_types=[
            pltpu.VMEM((per_w,), jnp.int32),
            pltpu.VMEM((chunk, d), jnp.int32),
            pltpu.VMEM((chunk, d), jnp.int32),
            pltpu.VMEM((nb, h, d), jnp.bfloat16),
            pltpu.VMEM((nb, h, d), jnp.bfloat16),
            pltpu.SemaphoreType.DMA,
            pltpu.SemaphoreType.DMA,
            pltpu.SemaphoreType.DMA,
            pltpu.SemaphoreType.DMA,
        ],
        compiler_params=pltpu.CompilerParams(
            use_tc_tiling_on_sc=False, needs_layout_passes=False),
    )
    def gather_kernel(table_hbm, idx_hbm, out_hbm, idx_v, g0, g1, s0, s1,
                      gsem0, gsem1, ssem0, ssem1):
        wid = lax.axis_index("s") * _NC + lax.axis_index("c")
        base = wid * per_w
        batch0 = wid * batches_per_w
        pltpu.sync_copy(idx_hbm.at[pl.ds(base, per_w)], idx_v)

        gbufs = (g0, g1)
        sbufs = (s0, s1)
        gsems = (gsem0, gsem1)
        ssems = (ssem0, ssem1)

        lane = lax.iota(jnp.int32, _LANES)
        even_lanes = [32 * k + 2 * lane for k in range(d // (2 * _LANES))]
        odd_lanes = [v + 1 for v in even_lanes]
        lo_mask = jnp.int32(0xFFFF)
        hi_mask = jnp.int32(-0x10000)

        def start_gather(i):
            return pltpu.async_copy(
                table_hbm.at[idx_v.at[pl.ds(i * chunk, chunk)]],
                gbufs[i % 2], gsems[i % 2])

        def stage_and_store(i):
            g = gbufs[i % 2]
            s = sbufs[i % 2]

            def copy_row(r, _):
                q = r // h
                row = r - q * h
                rsplat = jnp.full((_LANES,), r, jnp.int32)
                for k in range(d // (2 * _LANES)):
                    e = plsc.load_gather(g, [rsplat, even_lanes[k]])
                    o = plsc.load_gather(g, [rsplat, odd_lanes[k]])
                    w = (e & lo_mask) | (o & hi_mask)
                    s[q, row, pl.ds(k * 2 * _LANES, 2 * _LANES)] = (
                        plsc.bitcast(w, jnp.bfloat16))
                return ()

            lax.fori_loop(0, chunk, copy_row, (), unroll=2)
            return pltpu.async_copy(
                s, out_hbm.at[pl.ds(batch0 + i * nb, nb)], ssems[i % 2])

        gather_pend = [None] * n_chunks
        store_pend = [None] * n_chunks
        gather_pend[0] = start_gather(0)
        for i in range(n_chunks):
            if i + 1 < n_chunks:
                gather_pend[i + 1] = start_gather(i + 1)
            gather_pend[i].wait()
            if i >= 2:
                store_pend[i - 2].wait()
            store_pend[i] = stage_and_store(i)
        for i in range(max(n_chunks - 2, 0), n_chunks):
            store_pend[i].wait()

    return gather_kernel


def kernel(input, weight, weight_scaler):
    b, h = input.shape
    n, d = weight.shape
    table32 = _dequant_table(weight, weight_scaler)
    flat_idx = input.reshape(-1)
    gather = _make_gather(b, h, d, n, nb=1)
    return gather(table32, flat_idx)


# speedup vs baseline: 8.4713x; 2.1835x over previous
import functools

import jax
import jax.numpy as jnp
from jax import lax
from jax.experimental import pallas as pl
from jax.experimental.pallas import tpu as pltpu
from jax.experimental.pallas import tpu_sc as plsc

_NC = 2
_NS = 16
_NW = _NC * _NS
_LANES = 16


def _dequant_body(w_ref, s_ref, o_ref):
    vals = w_ref[...].astype(jnp.bfloat16) * s_ref[...]
    u = pltpu.bitcast(vals, jnp.uint16).astype(jnp.uint32)
    o_ref[...] = (u | (u << 16)).astype(jnp.int32)


def _dequant_table(weight, scaler, blk=4000):
    n, d = weight.shape
    return pl.pallas_call(
        _dequant_body,
        grid=(n // blk,),
        in_specs=[
            pl.BlockSpec((blk, d), lambda i: (i, 0)),
            pl.BlockSpec((1, d), lambda i: (0, 0)),
        ],
        out_specs=pl.BlockSpec((blk, d), lambda i: (i, 0)),
        out_shape=jax.ShapeDtypeStruct((n, d), jnp.int32),
    )(weight, scaler.reshape(1, d))


def _make_gather(b, h, d, n, nb):
    num_idx = b * h
    per_w = num_idx // _NW
    batches_per_w = b // _NW
    chunk = nb * h
    hp = h // 2
    n_chunks = batches_per_w // nb
    assert b % _NW == 0 and batches_per_w % nb == 0 and h % 2 == 0

    @functools.partial(
        pl.kernel,
        out_type=jax.ShapeDtypeStruct((b, h, d), jnp.bfloat16),
        mesh=plsc.VectorSubcoreMesh(
            core_axis_name="c", subcore_axis_name="s",
            num_cores=_NC, num_subcores=_NS),
        scratch_types=[
            pltpu.VMEM((per_w,), jnp.int32),
            pltpu.VMEM((chunk, d), jnp.int32),
            pltpu.VMEM((chunk, d), jnp.int32),
            pltpu.VMEM((nb, hp, d), jnp.int32),
            pltpu.VMEM((nb, hp, d), jnp.int32),
            pltpu.SemaphoreType.DMA,
            pltpu.SemaphoreType.DMA,
            pltpu.SemaphoreType.DMA,
            pltpu.SemaphoreType.DMA,
        ],
        compiler_params=pltpu.CompilerParams(needs_layout_passes=False),
    )
    def gather_kernel(table_hbm, idx_hbm, out_hbm, idx_v, g0, g1, s0, s1,
                      gsem0, gsem1, ssem0, ssem1):
        wid = lax.axis_index("s") * _NC + lax.axis_index("c")
        base = wid * per_w
        batch0 = wid * batches_per_w
        out32 = out_hbm.bitcast(jnp.int32)
        pltpu.sync_copy(idx_hbm.at[pl.ds(base, per_w)], idx_v)

        gbufs = (g0, g1)
        sbufs = (s0, s1)
        gsems = (gsem0, gsem1)
        ssems = (ssem0, ssem1)

        lo_mask = jnp.int32(0xFFFF)

        def start_gather(i):
            return pltpu.async_copy(
                table_hbm.at[idx_v.at[pl.ds(i * chunk, chunk)]],
                gbufs[i % 2], gsems[i % 2])

        def stage_and_store(i):
            g = gbufs[i % 2]
            s = sbufs[i % 2]

            def pack_pair(p, _):
                q = p // hp
                row = p - q * hp
                for k in range(d // _LANES):
                    va = g[2 * p, pl.ds(k * _LANES, _LANES)]
                    vb = g[2 * p + 1, pl.ds(k * _LANES, _LANES)]
                    s[q, row, pl.ds(k * _LANES, _LANES)] = (
                        (va & lo_mask) | (vb << 16))
                return ()

            lax.fori_loop(0, nb * hp, pack_pair, (), unroll=2)
            return pltpu.async_copy(
                s, out32.at[pl.ds(batch0 + i * nb, nb)], ssems[i % 2])

        gather_pend = [None] * n_chunks
        store_pend = [None] * n_chunks
        gather_pend[0] = start_gather(0)
        for i in range(n_chunks):
            if i + 1 < n_chunks:
                gather_pend[i + 1] = start_gather(i + 1)
            gather_pend[i].wait()
            if i >= 2:
                store_pend[i - 2].wait()
            store_pend[i] = stage_and_store(i)
        for i in range(max(n_chunks - 2, 0), n_chunks):
            store_pend[i].wait()

    return gather_kernel


def kernel(input, weight, weight_scaler):
    b, h = input.shape
    n, d = weight.shape
    table32 = _dequant_table(weight, weight_scaler)
    flat_idx = input.reshape(-1)
    gather = _make_gather(b, h, d, n, nb=1)
    return gather(table32, flat_idx)


# speedup vs baseline: 8.7146x; 1.0287x over previous
import functools

import jax
import jax.numpy as jnp
from jax import lax
from jax.experimental import pallas as pl
from jax.experimental.pallas import tpu as pltpu
from jax.experimental.pallas import tpu_sc as plsc

_NC = 2
_NS = 16
_NW = _NC * _NS
_LANES = 16


def _dequant_body(w_ref, s_ref, o_ref):
    vals = w_ref[...].astype(jnp.bfloat16) * s_ref[...]
    u = pltpu.bitcast(vals, jnp.uint16).astype(jnp.uint32)
    o_ref[...] = (u | (u << 16)).astype(jnp.int32)


def _dequant_table(weight, scaler, blk=4000):
    n, d = weight.shape
    return pl.pallas_call(
        _dequant_body,
        grid=(n // blk,),
        in_specs=[
            pl.BlockSpec((blk, d), lambda i: (i, 0)),
            pl.BlockSpec((1, d), lambda i: (0, 0)),
        ],
        out_specs=pl.BlockSpec((blk, d), lambda i: (i, 0)),
        out_shape=jax.ShapeDtypeStruct((n, d), jnp.int32),
    )(weight, scaler.reshape(1, d))


def _make_gather(b, h, d, n, nb):
    num_idx = b * h
    per_w = num_idx // _NW
    batches_per_w = b // _NW
    chunk = nb * h
    hp = h // 2
    n_chunks = batches_per_w // nb
    assert b % _NW == 0 and batches_per_w % nb == 0 and h % 2 == 0
    assert nb == 1

    @functools.partial(
        pl.kernel,
        out_type=jax.ShapeDtypeStruct((b, h, d), jnp.bfloat16),
        mesh=plsc.VectorSubcoreMesh(
            core_axis_name="c", subcore_axis_name="s",
            num_cores=_NC, num_subcores=_NS),
        scratch_types=[
            pltpu.VMEM((per_w,), jnp.int32),
            pltpu.VMEM((chunk, d), jnp.int32),
            pltpu.VMEM((chunk, d), jnp.int32),
            pltpu.VMEM((nb, hp, d), jnp.int32),
            pltpu.VMEM((nb, hp, d), jnp.int32),
            pltpu.SemaphoreType.DMA,
            pltpu.SemaphoreType.DMA,
            pltpu.SemaphoreType.DMA,
            pltpu.SemaphoreType.DMA,
        ],
        compiler_params=pltpu.CompilerParams(needs_layout_passes=False),
    )
    def gather_kernel(table_hbm, idx_hbm, out_hbm, idx_v, g0, g1, s0, s1,
                      gsem0, gsem1, ssem0, ssem1):
        wid = lax.axis_index("s") * _NC + lax.axis_index("c")
        base = wid * per_w
        batch0 = wid * batches_per_w
        out32 = out_hbm.bitcast(jnp.int32)
        pltpu.sync_copy(idx_hbm.at[pl.ds(base, per_w)], idx_v)

        gbufs = (g0, g1)
        sbufs = (s0, s1)
        gsems = (gsem0, gsem1)
        ssems = (ssem0, ssem1)

        lo_mask = jnp.int32(0xFFFF)

        def gather_copy(i, bsel):
            return pltpu.make_async_copy(
                table_hbm.at[idx_v.at[pl.ds(i * chunk, chunk)]],
                gbufs[bsel], gsems[bsel])

        def store_copy(i, bsel):
            return pltpu.make_async_copy(
                sbufs[bsel], out32.at[pl.ds(batch0 + i * nb, nb)],
                ssems[bsel])

        def pack_chunk(bsel):
            g = gbufs[bsel]
            s = sbufs[bsel]

            def pack_pair(p, _):
                for k in range(d // _LANES):
                    va = g[2 * p, pl.ds(k * _LANES, _LANES)]
                    vb = g[2 * p + 1, pl.ds(k * _LANES, _LANES)]
                    s[0, p, pl.ds(k * _LANES, _LANES)] = (
                        (va & lo_mask) | (vb << 16))
                return ()

            lax.fori_loop(0, nb * hp, pack_pair, (), unroll=10)

        gather_copy(0, 0).start()

        def chunk_body(i, _):
            for bsel in range(2):
                @pl.when(lax.rem(i, 2) == bsel)
                def _():
                    @pl.when(i + 1 < n_chunks)
                    def _():
                        gather_copy(i + 1, 1 - bsel).start()
                    gather_copy(i, bsel).wait()
                    @pl.when(i >= 2)
                    def _():
                        store_copy(i - 2, bsel).wait()
                    pack_chunk(bsel)
                    store_copy(i, bsel).start()
            return ()

        lax.fori_loop(0, n_chunks, chunk_body, ())
        store_copy(n_chunks - 2, n_chunks % 2).wait()
        store_copy(n_chunks - 1, 1 - n_chunks % 2).wait()

    return gather_kernel


def kernel(input, weight, weight_scaler):
    b, h = input.shape
    n, d = weight.shape
    table32 = _dequant_table(weight, weight_scaler)
    flat_idx = input.reshape(-1)
    gather = _make_gather(b, h, d, n, nb=1)
    return gather(table32, flat_idx)


# speedup vs baseline: 14.8390x; 1.7028x over previous
import functools

import jax
import jax.numpy as jnp
from jax import lax
from jax.experimental import pallas as pl
from jax.experimental.pallas import tpu as pltpu
from jax.experimental.pallas import tpu_sc as plsc

_NC = 2
_NS = 16
_NW = _NC * _NS
_LANES = 16


def _dequant_body(w_ref, s_ref, o_ref):
    vals = w_ref[...].astype(jnp.bfloat16) * s_ref[...]
    u = pltpu.bitcast(vals, jnp.uint16).astype(jnp.uint32)
    o_ref[...] = (u | (u << 16)).astype(jnp.int32)


def _dequant_table(weight, scaler, blk=4000):
    n, d = weight.shape
    return pl.pallas_call(
        _dequant_body,
        grid=(n // blk,),
        in_specs=[
            pl.BlockSpec((blk, d), lambda i: (i, 0)),
            pl.BlockSpec((1, d), lambda i: (0, 0)),
        ],
        out_specs=pl.BlockSpec((blk, d), lambda i: (i, 0)),
        out_shape=jax.ShapeDtypeStruct((n, d), jnp.int32),
    )(weight, scaler.reshape(1, d))


def _make_gather(b, h, d, n, nb):
    num_idx = b * h
    per_w = num_idx // _NW
    batches_per_w = b // _NW
    chunk = nb * h
    hp = h // 2
    n_chunks = batches_per_w // nb
    assert b % _NW == 0 and batches_per_w % nb == 0 and h % 2 == 0
    assert nb == 1

    @functools.partial(
        pl.kernel,
        out_type=jax.ShapeDtypeStruct((b, h, d), jnp.bfloat16),
        mesh=plsc.VectorSubcoreMesh(
            core_axis_name="c", subcore_axis_name="s",
            num_cores=_NC, num_subcores=_NS),
        scratch_types=[
            pltpu.VMEM((per_w,), jnp.int32),
            pltpu.VMEM((chunk, d), jnp.int32),
            pltpu.VMEM((chunk, d), jnp.int32),
            pltpu.VMEM((nb, hp, d), jnp.int32),
            pltpu.VMEM((nb, hp, d), jnp.int32),
            pltpu.SemaphoreType.DMA,
            pltpu.SemaphoreType.DMA,
            pltpu.SemaphoreType.DMA,
            pltpu.SemaphoreType.DMA,
        ],
        compiler_params=pltpu.CompilerParams(needs_layout_passes=False),
    )
    def gather_kernel(table_hbm, idx_hbm, out_hbm, idx_v, g0, g1, s0, s1,
                      gsem0, gsem1, ssem0, ssem1):
        wid = lax.axis_index("s") * _NC + lax.axis_index("c")
        base = wid * per_w
        batch0 = wid * batches_per_w
        out32 = out_hbm.bitcast(jnp.int32)
        pltpu.sync_copy(idx_hbm.at[pl.ds(base, per_w)], idx_v)

        gbufs = (g0, g1)
        sbufs = (s0, s1)
        gsems = (gsem0, gsem1)
        ssems = (ssem0, ssem1)

        lo_mask = jnp.int32(0xFFFF)

        def gather_copy(i, bsel):
            return pltpu.make_async_copy(
                table_hbm.at[idx_v.at[pl.ds(i * chunk, chunk)]],
                gbufs[bsel], gsems[bsel])

        def store_copy(i, bsel):
            return pltpu.make_async_copy(
                sbufs[bsel], out32.at[pl.ds(batch0 + i * nb, nb)],
                ssems[bsel])

        def pack_chunk(bsel):
            g = gbufs[bsel]
            s = sbufs[bsel]

            @plsc.parallel_loop(0, nb * hp, unroll=4)
            def _pack(p):
                for k in range(d // _LANES):
                    va = g[2 * p, pl.ds(k * _LANES, _LANES)]
                    vb = g[2 * p + 1, pl.ds(k * _LANES, _LANES)]
                    s[0, p, pl.ds(k * _LANES, _LANES)] = (
                        (va & lo_mask) | (vb << 16))

        gather_copy(0, 0).start()

        def chunk_body(i, _):
            for bsel in range(2):
                @pl.when(lax.rem(i, 2) == bsel)
                def _():
                    @pl.when(i + 1 < n_chunks)
                    def _():
                        gather_copy(i + 1, 1 - bsel).start()
                    gather_copy(i, bsel).wait()
                    @pl.when(i >= 2)
                    def _():
                        store_copy(i - 2, bsel).wait()
                    pack_chunk(bsel)
                    store_copy(i, bsel).start()
            return ()

        lax.fori_loop(0, n_chunks, chunk_body, ())
        store_copy(n_chunks - 2, n_chunks % 2).wait()
        store_copy(n_chunks - 1, 1 - n_chunks % 2).wait()

    return gather_kernel


def kernel(input, weight, weight_scaler):
    b, h = input.shape
    n, d = weight.shape
    table32 = _dequant_table(weight, weight_scaler)
    flat_idx = input.reshape(-1)
    gather = _make_gather(b, h, d, n, nb=1)
    return gather(table32, flat_idx)
